# baseline (speedup 1.0000x reference)
import functools

import jax
import jax.numpy as jnp
from jax import lax
from jax.experimental import pallas as pl
from jax.experimental.pallas import tpu as pltpu
from jax.experimental.pallas import tpu_sc as plsc

_B = 16384
_D = 64
_DP = 2 * _D
_NC = 2
_NS = 16
_NW = _NC * _NS
_BPW = _B // _NW
_IDXW = 128
_NCHUNK = _BPW // _IDXW

_TB = 1024


def _build_gather():
    mesh = plsc.VectorSubcoreMesh(core_axis_name="c", subcore_axis_name="s")

    @functools.partial(
        pl.kernel,
        mesh=mesh,
        out_type=(
            jax.ShapeDtypeStruct((_B, _DP), jnp.float32),
            jax.ShapeDtypeStruct((_B, _DP), jnp.float32),
        ),
        scratch_types=[
            pltpu.VMEM((_NCHUNK, _IDXW), jnp.int32),
            pltpu.VMEM((_NCHUNK, _IDXW), jnp.int32),
            pltpu.VMEM((_IDXW, _DP), jnp.float32),
            pltpu.VMEM((_IDXW, _DP), jnp.float32),
            pltpu.VMEM((_IDXW, _DP), jnp.float32),
            pltpu.VMEM((_IDXW, _DP), jnp.float32),
            pltpu.SemaphoreType.DMA,
            pltpu.SemaphoreType.DMA,
        ],
    )
    def gather(user_hbm, movie_hbm, ut_hbm, mt_hbm, ue_hbm, me_hbm,
               uidx, midx, ub0, ub1, mb0, mb1, sem0, sem1):
        wid = lax.axis_index("s") * _NC + lax.axis_index("c")
        base = wid * _BPW
        pltpu.sync_copy(user_hbm.at[wid], uidx)
        pltpu.sync_copy(movie_hbm.at[wid], midx)
        ubufs, mbufs, sems = (ub0, ub1), (mb0, mb1), (sem0, sem1)
        cu = [None] * _NCHUNK
        cm = [None] * _NCHUNK
        cu[0] = pltpu.async_copy(ut_hbm.at[uidx.at[0]], ubufs[0], sems[0])
        cm[0] = pltpu.async_copy(mt_hbm.at[midx.at[0]], mbufs[0], sems[0])
        for j in range(_NCHUNK):
            if j + 1 < _NCHUNK:
                k = (j + 1) % 2
                cu[j + 1] = pltpu.async_copy(
                    ut_hbm.at[uidx.at[j + 1]], ubufs[k], sems[k])
                cm[j + 1] = pltpu.async_copy(
                    mt_hbm.at[midx.at[j + 1]], mbufs[k], sems[k])
            cu[j].wait()
            cm[j].wait()
            dst = pl.ds(base + j * _IDXW, _IDXW)
            pltpu.sync_copy(ubufs[j % 2], ue_hbm.at[dst])
            pltpu.sync_copy(mbufs[j % 2], me_hbm.at[dst])

    return gather


def _mlp_body(ue, me, up, mp, w1u, w1m, b1, w2, b2, w3, b3, w4, b4, w5, b5,
              out):
    f32 = jnp.float32
    lane = lax.broadcasted_iota(jnp.int32, (_TB, _DP), 1)
    left = (lane < _D).astype(f32)
    umask = jnp.where(up[...] > 0.5, 1.0 - left, left)
    mmask = jnp.where(mp[...] > 0.5, 1.0 - left, left)
    x = jnp.dot(ue[...] * umask, w1u[...], preferred_element_type=f32)
    x = x + jnp.dot(me[...] * mmask, w1m[...], preferred_element_type=f32)
    x = jnp.maximum(x + b1[...], 0.0)
    x = jnp.maximum(jnp.dot(x, w2[...], preferred_element_type=f32) + b2[...], 0.0)
    x = jnp.maximum(jnp.dot(x, w3[...], preferred_element_type=f32) + b3[...], 0.0)
    x = jnp.maximum(jnp.dot(x, w4[...], preferred_element_type=f32) + b4[...], 0.0)
    out[...] = jnp.dot(x, w5[...], preferred_element_type=f32) + b5[...]


def _full(shape):
    return pl.BlockSpec(shape, lambda i: (0,) * len(shape))


def _mlp(ue, me, up, mp, w1u, w1m, b1, w2, b2, w3, b3, w4, b4, w5, b5):
    nblk = _B // _TB
    return pl.pallas_call(
        _mlp_body,
        grid=(nblk,),
        in_specs=[
            pl.BlockSpec((_TB, _DP), lambda i: (i, 0)),
            pl.BlockSpec((_TB, _DP), lambda i: (i, 0)),
            pl.BlockSpec((_TB, 1), lambda i: (i, 0)),
            pl.BlockSpec((_TB, 1), lambda i: (i, 0)),
            _full(w1u.shape), _full(w1m.shape), _full(b1.shape),
            _full(w2.shape), _full(b2.shape),
            _full(w3.shape), _full(b3.shape),
            _full(w4.shape), _full(b4.shape),
            _full(w5.shape), _full(b5.shape),
        ],
        out_specs=pl.BlockSpec((_TB, 1), lambda i: (i, 0)),
        out_shape=jax.ShapeDtypeStruct((_B, 1), jnp.float32),
    )(ue, me, up, mp, w1u, w1m, b1, w2, b2, w3, b3, w4, b4, w5, b5)


def kernel(user, movie, user_table, movie_table,
           W1, b1, W2, b2, W3, b3, W4, b4, W5, b5):
    user = user.astype(jnp.int32)
    movie = movie.astype(jnp.int32)
    urow = (user >> 1).reshape(_NW, _NCHUNK, _IDXW)
    mrow = (movie >> 1).reshape(_NW, _NCHUNK, _IDXW)
    up = (user & 1).astype(jnp.float32).reshape(_B, 1)
    mp = (movie & 1).astype(jnp.float32).reshape(_B, 1)
    tu = user_table.reshape(user_table.shape[0] // 2, _DP)
    tm = movie_table.reshape(movie_table.shape[0] // 2, _DP)
    ue, me = _build_gather()(urow, mrow, tu, tm)
    w1u = jnp.concatenate([W1[:_D], W1[:_D]], axis=0)
    w1m = jnp.concatenate([W1[_D:], W1[_D:]], axis=0)
    return _mlp(
        ue, me, up, mp,
        w1u, w1m, b1.reshape(1, -1),
        W2, b2.reshape(1, -1),
        W3, b3.reshape(1, -1),
        W4, b4.reshape(1, -1),
        W5, b5.reshape(1, -1),
    )


# speedup vs baseline: 1.5736x; 1.5736x over previous
import functools

import jax
import jax.numpy as jnp
from jax import lax
from jax.experimental import pallas as pl
from jax.experimental.pallas import tpu as pltpu
from jax.experimental.pallas import tpu_sc as plsc

_B = 16384
_D = 64
_DP = 2 * _D
_V = 1000000
_VP = _V // 2
_NC = 2
_NS = 16
_NW = _NC * _NS
_BPW = _B // _NW
_IDXW = 128
_NCHUNK = _BPW // _IDXW

_CL = 4096
_CSUB = 256
_TB = 1024


def _conv_body(x_ref, e_ref, o_ref, out_ref):
    i = pl.program_id(0)
    x = x_ref[...]
    lane = lax.broadcasted_iota(jnp.int32, (_D, _CL), 1) + i * _CL
    x = jnp.where(lane < _V, x, 0.0)
    for j in range(_CL // _CSUB):
        xj = x[:, j * _CSUB:(j + 1) * _CSUB]
        ej = e_ref[...]
        oj = o_ref[...]
        xe = lax.dot_general(ej, xj, (((0,), (1,)), ((), ())),
                             preferred_element_type=jnp.float32)
        xo = lax.dot_general(oj, xj, (((0,), (1,)), ((), ())),
                             preferred_element_type=jnp.float32)
        r0 = j * (_CSUB // 2)
        out_ref[r0:r0 + _CSUB // 2, 0:_D] = xe
        out_ref[r0:r0 + _CSUB // 2, _D:_DP] = xo


def _make_eo():
    c = jnp.arange(_CSUB)[:, None]
    p = jnp.arange(_CSUB // 2)[None, :]
    e = (c == 2 * p).astype(jnp.float32)
    o = (c == 2 * p + 1).astype(jnp.float32)
    return e, o


def _convert(table_t):
    nblk = (_V + _CL - 1) // _CL
    e, o = _make_eo()
    return pl.pallas_call(
        _conv_body,
        grid=(nblk,),
        in_specs=[
            pl.BlockSpec((_D, _CL), lambda i: (0, i)),
            pl.BlockSpec((_CSUB, _CSUB // 2), lambda i: (0, 0)),
            pl.BlockSpec((_CSUB, _CSUB // 2), lambda i: (0, 0)),
        ],
        out_specs=pl.BlockSpec((_CL // 2, _DP), lambda i: (i, 0)),
        out_shape=jax.ShapeDtypeStruct((_VP, _DP), jnp.float32),
    )(table_t, e, o)


def _build_gather():
    mesh = plsc.VectorSubcoreMesh(core_axis_name="c", subcore_axis_name="s")

    @functools.partial(
        pl.kernel,
        mesh=mesh,
        out_type=(
            jax.ShapeDtypeStruct((_B, _DP), jnp.float32),
            jax.ShapeDtypeStruct((_B, _DP), jnp.float32),
        ),
        scratch_types=[
            pltpu.VMEM((_NCHUNK, _IDXW), jnp.int32),
            pltpu.VMEM((_NCHUNK, _IDXW), jnp.int32),
            pltpu.VMEM((_IDXW, _DP), jnp.float32),
            pltpu.VMEM((_IDXW, _DP), jnp.float32),
            pltpu.VMEM((_IDXW, _DP), jnp.float32),
            pltpu.VMEM((_IDXW, _DP), jnp.float32),
            pltpu.SemaphoreType.DMA,
            pltpu.SemaphoreType.DMA,
        ],
    )
    def gather(user_hbm, movie_hbm, ut_hbm, mt_hbm, ue_hbm, me_hbm,
               uidx, midx, ub0, ub1, mb0, mb1, sem0, sem1):
        wid = lax.axis_index("s") * _NC + lax.axis_index("c")
        base = wid * _BPW
        pltpu.sync_copy(user_hbm.at[wid], uidx)
        pltpu.sync_copy(movie_hbm.at[wid], midx)
        ubufs, mbufs, sems = (ub0, ub1), (mb0, mb1), (sem0, sem1)
        cu = [None] * _NCHUNK
        cm = [None] * _NCHUNK
        cu[0] = pltpu.async_copy(ut_hbm.at[uidx.at[0]], ubufs[0], sems[0])
        cm[0] = pltpu.async_copy(mt_hbm.at[midx.at[0]], mbufs[0], sems[0])
        for j in range(_NCHUNK):
            if j + 1 < _NCHUNK:
                k = (j + 1) % 2
                cu[j + 1] = pltpu.async_copy(
                    ut_hbm.at[uidx.at[j + 1]], ubufs[k], sems[k])
                cm[j + 1] = pltpu.async_copy(
                    mt_hbm.at[midx.at[j + 1]], mbufs[k], sems[k])
            cu[j].wait()
            cm[j].wait()
            dst = pl.ds(base + j * _IDXW, _IDXW)
            pltpu.sync_copy(ubufs[j % 2], ue_hbm.at[dst])
            pltpu.sync_copy(mbufs[j % 2], me_hbm.at[dst])

    return gather


def _mlp_body(ue, me, up, mp, w1u, w1m, b1, w2, b2, w3, b3, w4, b4, w5, b5,
              out):
    f32 = jnp.float32
    lane = lax.broadcasted_iota(jnp.int32, (_TB, _DP), 1)
    left = (lane < _D).astype(f32)
    umask = jnp.where(up[...] > 0.5, 1.0 - left, left)
    mmask = jnp.where(mp[...] > 0.5, 1.0 - left, left)
    x = jnp.dot(ue[...] * umask, w1u[...], preferred_element_type=f32)
    x = x + jnp.dot(me[...] * mmask, w1m[...], preferred_element_type=f32)
    x = jnp.maximum(x + b1[...], 0.0)
    x = jnp.maximum(jnp.dot(x, w2[...], preferred_element_type=f32) + b2[...], 0.0)
    x = jnp.maximum(jnp.dot(x, w3[...], preferred_element_type=f32) + b3[...], 0.0)
    x = jnp.maximum(jnp.dot(x, w4[...], preferred_element_type=f32) + b4[...], 0.0)
    out[...] = jnp.dot(x, w5[...], preferred_element_type=f32) + b5[...]


def _full(shape):
    return pl.BlockSpec(shape, lambda i: (0,) * len(shape))


def _mlp(ue, me, up, mp, w1u, w1m, b1, w2, b2, w3, b3, w4, b4, w5, b5):
    nblk = _B // _TB
    return pl.pallas_call(
        _mlp_body,
        grid=(nblk,),
        in_specs=[
            pl.BlockSpec((_TB, _DP), lambda i: (i, 0)),
            pl.BlockSpec((_TB, _DP), lambda i: (i, 0)),
            pl.BlockSpec((_TB, 1), lambda i: (i, 0)),
            pl.BlockSpec((_TB, 1), lambda i: (i, 0)),
            _full(w1u.shape), _full(w1m.shape), _full(b1.shape),
            _full(w2.shape), _full(b2.shape),
            _full(w3.shape), _full(b3.shape),
            _full(w4.shape), _full(b4.shape),
            _full(w5.shape), _full(b5.shape),
        ],
        out_specs=pl.BlockSpec((_TB, 1), lambda i: (i, 0)),
        out_shape=jax.ShapeDtypeStruct((_B, 1), jnp.float32),
    )(ue, me, up, mp, w1u, w1m, b1, w2, b2, w3, b3, w4, b4, w5, b5)


def kernel(user, movie, user_table, movie_table,
           W1, b1, W2, b2, W3, b3, W4, b4, W5, b5):
    user = user.astype(jnp.int32)
    movie = movie.astype(jnp.int32)
    urow = (user >> 1).reshape(_NW, _NCHUNK, _IDXW)
    mrow = (movie >> 1).reshape(_NW, _NCHUNK, _IDXW)
    up = (user & 1).astype(jnp.float32).reshape(_B, 1)
    mp = (movie & 1).astype(jnp.float32).reshape(_B, 1)
    tu = _convert(user_table.T)
    tm = _convert(movie_table.T)
    ue, me = _build_gather()(urow, mrow, tu, tm)
    w1u = jnp.concatenate([W1[:_D], W1[:_D]], axis=0)
    w1m = jnp.concatenate([W1[_D:], W1[_D:]], axis=0)
    return _mlp(
        ue, me, up, mp,
        w1u, w1m, b1.reshape(1, -1),
        W2, b2.reshape(1, -1),
        W3, b3.reshape(1, -1),
        W4, b4.reshape(1, -1),
        W5, b5.reshape(1, -1),
    )


# speedup vs baseline: 1.9691x; 1.2513x over previous
import functools

import jax
import jax.numpy as jnp
from jax import lax
from jax.experimental import pallas as pl
from jax.experimental.pallas import tpu as pltpu
from jax.experimental.pallas import tpu_sc as plsc

_B = 16384
_D = 64
_DP = 2 * _D
_V = 1000000
_VP = _V // 2
_NC = 2
_NS = 16
_NW = _NC * _NS
_BPW = _B // _NW
_IDXW = 128
_NCHUNK = _BPW // _IDXW

_CL = 8192
_TB = 1024


_CH = _CL // 2
_NBLK = (_V + _CL - 1) // _CL


def _conv_body(x_ref, out_ref):
    x = x_ref[...]
    q = _CL // 4
    parts = [lax.transpose(x[:, k * q:(k + 1) * q], (1, 0))
             for k in range(4)]
    out_ref[0:q, :] = jnp.concatenate(parts[0:2], axis=1)
    out_ref[q:2 * q, :] = jnp.concatenate(parts[2:4], axis=1)


def _convert(table_t):
    return pl.pallas_call(
        _conv_body,
        grid=(_NBLK,),
        in_specs=[pl.BlockSpec((_D, _CL), lambda i: (0, i))],
        out_specs=pl.BlockSpec((_CH, _DP), lambda i: (i, 0)),
        out_shape=jax.ShapeDtypeStruct((_NBLK * _CH, _DP), jnp.float32),
    )(table_t)


def _build_gather():
    mesh = plsc.VectorSubcoreMesh(core_axis_name="c", subcore_axis_name="s")

    @functools.partial(
        pl.kernel,
        mesh=mesh,
        out_type=(
            jax.ShapeDtypeStruct((_B, _DP), jnp.float32),
            jax.ShapeDtypeStruct((_B, _DP), jnp.float32),
        ),
        scratch_types=[
            pltpu.VMEM((_NCHUNK, _IDXW), jnp.int32),
            pltpu.VMEM((_NCHUNK, _IDXW), jnp.int32),
            pltpu.VMEM((_IDXW, _DP), jnp.float32),
            pltpu.VMEM((_IDXW, _DP), jnp.float32),
            pltpu.VMEM((_IDXW, _DP), jnp.float32),
            pltpu.VMEM((_IDXW, _DP), jnp.float32),
            pltpu.SemaphoreType.DMA,
            pltpu.SemaphoreType.DMA,
        ],
    )
    def gather(user_hbm, movie_hbm, ut_hbm, mt_hbm, ue_hbm, me_hbm,
               uidx, midx, ub0, ub1, mb0, mb1, sem0, sem1):
        wid = lax.axis_index("s") * _NC + lax.axis_index("c")
        base = wid * _BPW
        pltpu.sync_copy(user_hbm.at[wid], uidx)
        pltpu.sync_copy(movie_hbm.at[wid], midx)
        ubufs, mbufs, sems = (ub0, ub1), (mb0, mb1), (sem0, sem1)
        cu = [None] * _NCHUNK
        cm = [None] * _NCHUNK
        cu[0] = pltpu.async_copy(ut_hbm.at[uidx.at[0]], ubufs[0], sems[0])
        cm[0] = pltpu.async_copy(mt_hbm.at[midx.at[0]], mbufs[0], sems[0])
        for j in range(_NCHUNK):
            if j + 1 < _NCHUNK:
                k = (j + 1) % 2
                cu[j + 1] = pltpu.async_copy(
                    ut_hbm.at[uidx.at[j + 1]], ubufs[k], sems[k])
                cm[j + 1] = pltpu.async_copy(
                    mt_hbm.at[midx.at[j + 1]], mbufs[k], sems[k])
            cu[j].wait()
            cm[j].wait()
            dst = pl.ds(base + j * _IDXW, _IDXW)
            pltpu.sync_copy(ubufs[j % 2], ue_hbm.at[dst])
            pltpu.sync_copy(mbufs[j % 2], me_hbm.at[dst])

    return gather


def _mlp_body(ue, me, up, mp, w1u, w1m, b1, w2, b2, w3, b3, w4, b4, w5, b5,
              out):
    f32 = jnp.float32
    lane = lax.broadcasted_iota(jnp.int32, (_TB, _DP), 1)
    left = (lane < _D).astype(f32)
    umask = jnp.where(up[...] > 0.5, 1.0 - left, left)
    mmask = jnp.where(mp[...] > 0.5, 1.0 - left, left)
    x = jnp.dot(ue[...] * umask, w1u[...], preferred_element_type=f32)
    x = x + jnp.dot(me[...] * mmask, w1m[...], preferred_element_type=f32)
    x = jnp.maximum(x + b1[...], 0.0)
    x = jnp.maximum(jnp.dot(x, w2[...], preferred_element_type=f32) + b2[...], 0.0)
    x = jnp.maximum(jnp.dot(x, w3[...], preferred_element_type=f32) + b3[...], 0.0)
    x = jnp.maximum(jnp.dot(x, w4[...], preferred_element_type=f32) + b4[...], 0.0)
    out[...] = jnp.dot(x, w5[...], preferred_element_type=f32) + b5[...]


def _full(shape):
    return pl.BlockSpec(shape, lambda i: (0,) * len(shape))


def _mlp(ue, me, up, mp, w1u, w1m, b1, w2, b2, w3, b3, w4, b4, w5, b5):
    nblk = _B // _TB
    return pl.pallas_call(
        _mlp_body,
        grid=(nblk,),
        in_specs=[
            pl.BlockSpec((_TB, _DP), lambda i: (i, 0)),
            pl.BlockSpec((_TB, _DP), lambda i: (i, 0)),
            pl.BlockSpec((_TB, 1), lambda i: (i, 0)),
            pl.BlockSpec((_TB, 1), lambda i: (i, 0)),
            _full(w1u.shape), _full(w1m.shape), _full(b1.shape),
            _full(w2.shape), _full(b2.shape),
            _full(w3.shape), _full(b3.shape),
            _full(w4.shape), _full(b4.shape),
            _full(w5.shape), _full(b5.shape),
        ],
        out_specs=pl.BlockSpec((_TB, 1), lambda i: (i, 0)),
        out_shape=jax.ShapeDtypeStruct((_B, 1), jnp.float32),
    )(ue, me, up, mp, w1u, w1m, b1, w2, b2, w3, b3, w4, b4, w5, b5)


def kernel(user, movie, user_table, movie_table,
           W1, b1, W2, b2, W3, b3, W4, b4, W5, b5):
    user = user.astype(jnp.int32)
    movie = movie.astype(jnp.int32)
    urow = (((user >> 12) << 11) + (user & 2047)).reshape(
        _NW, _NCHUNK, _IDXW)
    mrow = (((movie >> 12) << 11) + (movie & 2047)).reshape(
        _NW, _NCHUNK, _IDXW)
    up = ((user >> 11) & 1).astype(jnp.float32).reshape(_B, 1)
    mp = ((movie >> 11) & 1).astype(jnp.float32).reshape(_B, 1)
    tu = _convert(user_table.T)
    tm = _convert(movie_table.T)
    ue, me = _build_gather()(urow, mrow, tu, tm)
    w1u = jnp.concatenate([W1[:_D], W1[:_D]], axis=0)
    w1m = jnp.concatenate([W1[_D:], W1[_D:]], axis=0)
    return _mlp(
        ue, me, up, mp,
        w1u, w1m, b1.reshape(1, -1),
        W2, b2.reshape(1, -1),
        W3, b3.reshape(1, -1),
        W4, b4.reshape(1, -1),
        W5, b5.reshape(1, -1),
    )


# speedup vs baseline: 2.3482x; 1.1926x over previous
import functools

import jax
import jax.numpy as jnp
from jax import lax
from jax.experimental import pallas as pl
from jax.experimental.pallas import tpu as pltpu
from jax.experimental.pallas import tpu_sc as plsc

_B = 16384
_D = 64
_DP = 2 * _D
_V = 1000000
_VP = _V // 2
_NC = 2
_NS = 16
_NW = _NC * _NS
_BPW = _B // _NW
_IDXW = 128
_NCHUNK = _BPW // _IDXW

_CL = 8192
_TB = 1024


_CH = _CL // 2
_NBLK = (_V + _CL - 1) // _CL


def _conv_one(x, out_ref):
    q = _CL // 4
    parts = [lax.transpose(x[:, k * q:(k + 1) * q], (1, 0))
             for k in range(4)]
    out_ref[0:q, :] = jnp.concatenate(parts[0:2], axis=1)
    out_ref[q:2 * q, :] = jnp.concatenate(parts[2:4], axis=1)


def _conv_body(xu_ref, xm_ref, outu_ref, outm_ref):
    _conv_one(xu_ref[...], outu_ref)
    _conv_one(xm_ref[...], outm_ref)


def _convert(ut_t, mt_t):
    spec_in = pl.BlockSpec((_D, _CL), lambda i: (0, i))
    spec_out = pl.BlockSpec((_CH, _DP), lambda i: (i, 0))
    oshape = jax.ShapeDtypeStruct((_NBLK * _CH, _DP), jnp.float32)
    return pl.pallas_call(
        _conv_body,
        grid=(_NBLK,),
        in_specs=[spec_in, spec_in],
        out_specs=[spec_out, spec_out],
        out_shape=[oshape, oshape],
    )(ut_t, mt_t)


def _build_gather():
    mesh = plsc.VectorSubcoreMesh(core_axis_name="c", subcore_axis_name="s")

    @functools.partial(
        pl.kernel,
        mesh=mesh,
        out_type=(
            jax.ShapeDtypeStruct((_B, _DP), jnp.float32),
            jax.ShapeDtypeStruct((_B, _DP), jnp.float32),
        ),
        scratch_types=[
            pltpu.VMEM((_NCHUNK, _IDXW), jnp.int32),
            pltpu.VMEM((_NCHUNK, _IDXW), jnp.int32),
            pltpu.VMEM((_IDXW, _DP), jnp.float32),
            pltpu.VMEM((_IDXW, _DP), jnp.float32),
            pltpu.VMEM((_IDXW, _DP), jnp.float32),
            pltpu.VMEM((_IDXW, _DP), jnp.float32),
            pltpu.SemaphoreType.DMA,
            pltpu.SemaphoreType.DMA,
        ],
    )
    def gather(user_hbm, movie_hbm, ut_hbm, mt_hbm, ue_hbm, me_hbm,
               uidx, midx, ub0, ub1, mb0, mb1, sem0, sem1):
        wid = lax.axis_index("s") * _NC + lax.axis_index("c")
        base = wid * _BPW
        pltpu.sync_copy(user_hbm.at[wid], uidx)
        pltpu.sync_copy(movie_hbm.at[wid], midx)
        ubufs, mbufs, sems = (ub0, ub1), (mb0, mb1), (sem0, sem1)
        cu = [None] * _NCHUNK
        cm = [None] * _NCHUNK
        cu[0] = pltpu.async_copy(ut_hbm.at[uidx.at[0]], ubufs[0], sems[0])
        cm[0] = pltpu.async_copy(mt_hbm.at[midx.at[0]], mbufs[0], sems[0])
        for j in range(_NCHUNK):
            if j + 1 < _NCHUNK:
                k = (j + 1) % 2
                cu[j + 1] = pltpu.async_copy(
                    ut_hbm.at[uidx.at[j + 1]], ubufs[k], sems[k])
                cm[j + 1] = pltpu.async_copy(
                    mt_hbm.at[midx.at[j + 1]], mbufs[k], sems[k])
            cu[j].wait()
            cm[j].wait()
            dst = pl.ds(base + j * _IDXW, _IDXW)
            pltpu.sync_copy(ubufs[j % 2], ue_hbm.at[dst])
            pltpu.sync_copy(mbufs[j % 2], me_hbm.at[dst])

    return gather


def _mlp_body(ue, me, up, mp, w1u, w1m, b1, w2, b2, w3, b3, w4, b4, w5, b5,
              out):
    f32 = jnp.float32
    lane = lax.broadcasted_iota(jnp.int32, (_TB, _DP), 1)
    left = (lane < _D).astype(f32)
    umask = jnp.where(up[...] > 0.5, 1.0 - left, left)
    mmask = jnp.where(mp[...] > 0.5, 1.0 - left, left)
    x = jnp.dot(ue[...] * umask, w1u[...], preferred_element_type=f32)
    x = x + jnp.dot(me[...] * mmask, w1m[...], preferred_element_type=f32)
    x = jnp.maximum(x + b1[...], 0.0)
    x = jnp.maximum(jnp.dot(x, w2[...], preferred_element_type=f32) + b2[...], 0.0)
    x = jnp.maximum(jnp.dot(x, w3[...], preferred_element_type=f32) + b3[...], 0.0)
    x = jnp.maximum(jnp.dot(x, w4[...], preferred_element_type=f32) + b4[...], 0.0)
    out[...] = jnp.dot(x, w5[...], preferred_element_type=f32) + b5[...]


def _full(shape):
    return pl.BlockSpec(shape, lambda i: (0,) * len(shape))


def _mlp(ue, me, up, mp, w1u, w1m, b1, w2, b2, w3, b3, w4, b4, w5, b5):
    nblk = _B // _TB
    return pl.pallas_call(
        _mlp_body,
        grid=(nblk,),
        in_specs=[
            pl.BlockSpec((_TB, _DP), lambda i: (i, 0)),
            pl.BlockSpec((_TB, _DP), lambda i: (i, 0)),
            pl.BlockSpec((_TB, 1), lambda i: (i, 0)),
            pl.BlockSpec((_TB, 1), lambda i: (i, 0)),
            _full(w1u.shape), _full(w1m.shape), _full(b1.shape),
            _full(w2.shape), _full(b2.shape),
            _full(w3.shape), _full(b3.shape),
            _full(w4.shape), _full(b4.shape),
            _full(w5.shape), _full(b5.shape),
        ],
        out_specs=pl.BlockSpec((_TB, 1), lambda i: (i, 0)),
        out_shape=jax.ShapeDtypeStruct((_B, 1), jnp.float32),
    )(ue, me, up, mp, w1u, w1m, b1, w2, b2, w3, b3, w4, b4, w5, b5)


def kernel(user, movie, user_table, movie_table,
           W1, b1, W2, b2, W3, b3, W4, b4, W5, b5):
    user = user.astype(jnp.int32)
    movie = movie.astype(jnp.int32)
    urow = (((user >> 12) << 11) + (user & 2047)).reshape(
        _NW, _NCHUNK, _IDXW)
    mrow = (((movie >> 12) << 11) + (movie & 2047)).reshape(
        _NW, _NCHUNK, _IDXW)
    up = ((user >> 11) & 1).astype(jnp.float32).reshape(_B, 1)
    mp = ((movie >> 11) & 1).astype(jnp.float32).reshape(_B, 1)
    tu, tm = _convert(user_table.T, movie_table.T)
    ue, me = _build_gather()(urow, mrow, tu, tm)
    w1u = jnp.concatenate([W1[:_D], W1[:_D]], axis=0)
    w1m = jnp.concatenate([W1[_D:], W1[_D:]], axis=0)
    return _mlp(
        ue, me, up, mp,
        w1u, w1m, b1.reshape(1, -1),
        W2, b2.reshape(1, -1),
        W3, b3.reshape(1, -1),
        W4, b4.reshape(1, -1),
        W5, b5.reshape(1, -1),
    )


# speedup vs baseline: 2.9283x; 1.2470x over previous
import functools

import jax
import jax.numpy as jnp
from jax import lax
from jax.experimental import pallas as pl
from jax.experimental.pallas import tpu as pltpu
from jax.experimental.pallas import tpu_sc as plsc

_B = 16384
_D = 64
_DP = 2 * _D
_V = 1000000
_VP = _V // 2
_NC = 2
_NS = 16
_NW = _NC * _NS
_BPW = _B // _NW
_IDXW = 128
_NCHUNK = _BPW // _IDXW

_CL = 8192
_TB = 1024


_CH = _CL // 2
_NBLK = (_V + _CL - 1) // _CL


def _pack(hi, lo):
    hb = lax.bitcast_convert_type(hi.astype(jnp.bfloat16), jnp.uint16)
    lb = lax.bitcast_convert_type(lo.astype(jnp.bfloat16), jnp.uint16)
    return ((hb.astype(jnp.int32) << 16) |
            lb.astype(jnp.int32)).astype(jnp.int32)


def _conv_one(x, out_ref):
    q = _CL // 4
    parts = [lax.transpose(x[:, k * q:(k + 1) * q], (1, 0))
             for k in range(4)]
    out_ref[:, 0:_D] = _pack(parts[0], parts[1])
    out_ref[:, _D:_DP] = _pack(parts[2], parts[3])


def _conv_body(xu_ref, xm_ref, outu_ref, outm_ref):
    _conv_one(xu_ref[...], outu_ref)
    _conv_one(xm_ref[...], outm_ref)


def _convert(ut_t, mt_t):
    spec_in = pl.BlockSpec((_D, _CL), lambda i: (0, i))
    spec_out = pl.BlockSpec((_CH // 2, _DP), lambda i: (i, 0))
    oshape = jax.ShapeDtypeStruct((_NBLK * _CH // 2, _DP), jnp.int32)
    return pl.pallas_call(
        _conv_body,
        grid=(_NBLK,),
        in_specs=[spec_in, spec_in],
        out_specs=[spec_out, spec_out],
        out_shape=[oshape, oshape],
    )(ut_t, mt_t)


def _build_gather():
    mesh = plsc.VectorSubcoreMesh(core_axis_name="c", subcore_axis_name="s")

    @functools.partial(
        pl.kernel,
        mesh=mesh,
        out_type=(
            jax.ShapeDtypeStruct((_B, _DP), jnp.int32),
            jax.ShapeDtypeStruct((_B, _DP), jnp.int32),
        ),
        scratch_types=[
            pltpu.VMEM((_NCHUNK, _IDXW), jnp.int32),
            pltpu.VMEM((_NCHUNK, _IDXW), jnp.int32),
            pltpu.VMEM((_IDXW, _DP), jnp.int32),
            pltpu.VMEM((_IDXW, _DP), jnp.int32),
            pltpu.VMEM((_IDXW, _DP), jnp.int32),
            pltpu.VMEM((_IDXW, _DP), jnp.int32),
            pltpu.SemaphoreType.DMA,
            pltpu.SemaphoreType.DMA,
        ],
    )
    def gather(user_hbm, movie_hbm, ut_hbm, mt_hbm, ue_hbm, me_hbm,
               uidx, midx, ub0, ub1, mb0, mb1, sem0, sem1):
        wid = lax.axis_index("s") * _NC + lax.axis_index("c")
        base = wid * _BPW
        pltpu.sync_copy(user_hbm.at[wid], uidx)
        pltpu.sync_copy(movie_hbm.at[wid], midx)
        ubufs, mbufs, sems = (ub0, ub1), (mb0, mb1), (sem0, sem1)
        cu = [None] * _NCHUNK
        cm = [None] * _NCHUNK
        cu[0] = pltpu.async_copy(ut_hbm.at[uidx.at[0]], ubufs[0], sems[0])
        cm[0] = pltpu.async_copy(mt_hbm.at[midx.at[0]], mbufs[0], sems[0])
        for j in range(_NCHUNK):
            if j + 1 < _NCHUNK:
                k = (j + 1) % 2
                cu[j + 1] = pltpu.async_copy(
                    ut_hbm.at[uidx.at[j + 1]], ubufs[k], sems[k])
                cm[j + 1] = pltpu.async_copy(
                    mt_hbm.at[midx.at[j + 1]], mbufs[k], sems[k])
            cu[j].wait()
            cm[j].wait()
            dst = pl.ds(base + j * _IDXW, _IDXW)
            pltpu.sync_copy(ubufs[j % 2], ue_hbm.at[dst])
            pltpu.sync_copy(mbufs[j % 2], me_hbm.at[dst])

    return gather


def _unpack_sel(g, half, side):
    f32 = jnp.float32
    hi = lax.bitcast_convert_type(g & jnp.int32(-65536), f32)
    lo = lax.bitcast_convert_type(g << 16, f32)
    plane = jnp.where(half[...] > 0.5, lo, hi)
    lane = lax.broadcasted_iota(jnp.int32, (_TB, _DP), 1)
    left = (lane < _D).astype(f32)
    smask = jnp.where(side[...] > 0.5, 1.0 - left, left)
    return plane * smask


def _mlp_body(ue, me, uh, usd, mh, msd, w1u, w1m, b1, w2, b2, w3, b3, w4, b4,
              w5, b5, out):
    f32 = jnp.float32
    x = jnp.dot(_unpack_sel(ue[...], uh, usd), w1u[...],
                preferred_element_type=f32)
    x = x + jnp.dot(_unpack_sel(me[...], mh, msd), w1m[...],
                    preferred_element_type=f32)
    x = jnp.maximum(x + b1[...], 0.0)
    x = jnp.maximum(jnp.dot(x, w2[...], preferred_element_type=f32) + b2[...], 0.0)
    x = jnp.maximum(jnp.dot(x, w3[...], preferred_element_type=f32) + b3[...], 0.0)
    x = jnp.maximum(jnp.dot(x, w4[...], preferred_element_type=f32) + b4[...], 0.0)
    out[...] = jnp.dot(x, w5[...], preferred_element_type=f32) + b5[...]


def _full(shape):
    return pl.BlockSpec(shape, lambda i: (0,) * len(shape))


def _mlp(ue, me, uh, usd, mh, msd,
         w1u, w1m, b1, w2, b2, w3, b3, w4, b4, w5, b5):
    nblk = _B // _TB
    row = pl.BlockSpec((_TB, _DP), lambda i: (i, 0))
    col = pl.BlockSpec((_TB, 1), lambda i: (i, 0))
    return pl.pallas_call(
        _mlp_body,
        grid=(nblk,),
        in_specs=[
            row, row, col, col, col, col,
            _full(w1u.shape), _full(w1m.shape), _full(b1.shape),
            _full(w2.shape), _full(b2.shape),
            _full(w3.shape), _full(b3.shape),
            _full(w4.shape), _full(b4.shape),
            _full(w5.shape), _full(b5.shape),
        ],
        out_specs=pl.BlockSpec((_TB, 1), lambda i: (i, 0)),
        out_shape=jax.ShapeDtypeStruct((_B, 1), jnp.float32),
    )(ue, me, uh, usd, mh, msd,
      w1u, w1m, b1, w2, b2, w3, b3, w4, b4, w5, b5)


def kernel(user, movie, user_table, movie_table,
           W1, b1, W2, b2, W3, b3, W4, b4, W5, b5):
    user = user.astype(jnp.int32)
    movie = movie.astype(jnp.int32)
    urow = (((user >> 13) << 11) + (user & 2047)).reshape(
        _NW, _NCHUNK, _IDXW)
    mrow = (((movie >> 13) << 11) + (movie & 2047)).reshape(
        _NW, _NCHUNK, _IDXW)
    f32 = jnp.float32
    usd = ((user >> 12) & 1).astype(f32).reshape(_B, 1)
    msd = ((movie >> 12) & 1).astype(f32).reshape(_B, 1)
    uh = ((user >> 11) & 1).astype(f32).reshape(_B, 1)
    mh = ((movie >> 11) & 1).astype(f32).reshape(_B, 1)
    tu, tm = _convert(user_table.T, movie_table.T)
    ue, me = _build_gather()(urow, mrow, tu, tm)
    w1u = jnp.concatenate([W1[:_D], W1[:_D]], axis=0)
    w1m = jnp.concatenate([W1[_D:], W1[_D:]], axis=0)
    return _mlp(
        ue, me, uh, usd, mh, msd,
        w1u, w1m, b1.reshape(1, -1),
        W2, b2.reshape(1, -1),
        W3, b3.reshape(1, -1),
        W4, b4.reshape(1, -1),
        W5, b5.reshape(1, -1),
    )


# speedup vs baseline: 3.2629x; 1.1143x over previous
import functools

import jax
import jax.numpy as jnp
from jax import lax
from jax.experimental import pallas as pl
from jax.experimental.pallas import tpu as pltpu
from jax.experimental.pallas import tpu_sc as plsc

_B = 16384
_D = 64
_DP = 2 * _D
_V = 1000000
_VP = _V // 2
_NC = 2
_NS = 16
_NW = _NC * _NS
_BPW = _B // _NW
_IDXW = 128
_NCHUNK = _BPW // _IDXW

_CL = 16384
_Q = _CL // 4
_QSH = _Q.bit_length() - 1
_TB = 1024


_CH = _CL // 2
_NBLK = (_V + _CL - 1) // _CL


def _pack(hi, lo):
    hb = lax.bitcast_convert_type(hi.astype(jnp.bfloat16), jnp.uint16)
    lb = lax.bitcast_convert_type(lo.astype(jnp.bfloat16), jnp.uint16)
    return ((hb.astype(jnp.int32) << 16) |
            lb.astype(jnp.int32)).astype(jnp.int32)


def _conv_one(x, out_ref):
    q = _CL // 4
    parts = [lax.transpose(x[:, k * q:(k + 1) * q], (1, 0))
             for k in range(4)]
    out_ref[:, 0:_D] = _pack(parts[0], parts[1])
    out_ref[:, _D:_DP] = _pack(parts[2], parts[3])


def _conv_body(xu_ref, xm_ref, outu_ref, outm_ref):
    _conv_one(xu_ref[...], outu_ref)
    _conv_one(xm_ref[...], outm_ref)


def _convert(ut_t, mt_t):
    spec_in = pl.BlockSpec((_D, _CL), lambda i: (0, i))
    spec_out = pl.BlockSpec((_CH // 2, _DP), lambda i: (i, 0))
    oshape = jax.ShapeDtypeStruct((_NBLK * _CH // 2, _DP), jnp.int32)
    return pl.pallas_call(
        _conv_body,
        grid=(_NBLK,),
        in_specs=[spec_in, spec_in],
        out_specs=[spec_out, spec_out],
        out_shape=[oshape, oshape],
    )(ut_t, mt_t)


def _build_gather():
    mesh = plsc.VectorSubcoreMesh(core_axis_name="c", subcore_axis_name="s")

    @functools.partial(
        pl.kernel,
        mesh=mesh,
        out_type=(
            jax.ShapeDtypeStruct((_B, _DP), jnp.int32),
            jax.ShapeDtypeStruct((_B, _DP), jnp.int32),
        ),
        scratch_types=[
            pltpu.VMEM((_NCHUNK, _IDXW), jnp.int32),
            pltpu.VMEM((_NCHUNK, _IDXW), jnp.int32),
            pltpu.VMEM((_IDXW, _DP), jnp.int32),
            pltpu.VMEM((_IDXW, _DP), jnp.int32),
            pltpu.VMEM((_IDXW, _DP), jnp.int32),
            pltpu.VMEM((_IDXW, _DP), jnp.int32),
            pltpu.SemaphoreType.DMA,
            pltpu.SemaphoreType.DMA,
        ],
    )
    def gather(user_hbm, movie_hbm, ut_hbm, mt_hbm, ue_hbm, me_hbm,
               uidx, midx, ub0, ub1, mb0, mb1, sem0, sem1):
        wid = lax.axis_index("s") * _NC + lax.axis_index("c")
        base = wid * _BPW
        pltpu.sync_copy(user_hbm.at[wid], uidx)
        pltpu.sync_copy(movie_hbm.at[wid], midx)
        ubufs, mbufs, sems = (ub0, ub1), (mb0, mb1), (sem0, sem1)
        cu = [None] * _NCHUNK
        cm = [None] * _NCHUNK
        cu[0] = pltpu.async_copy(ut_hbm.at[uidx.at[0]], ubufs[0], sems[0])
        cm[0] = pltpu.async_copy(mt_hbm.at[midx.at[0]], mbufs[0], sems[0])
        for j in range(_NCHUNK):
            if j + 1 < _NCHUNK:
                k = (j + 1) % 2
                cu[j + 1] = pltpu.async_copy(
                    ut_hbm.at[uidx.at[j + 1]], ubufs[k], sems[k])
                cm[j + 1] = pltpu.async_copy(
                    mt_hbm.at[midx.at[j + 1]], mbufs[k], sems[k])
            cu[j].wait()
            cm[j].wait()
            dst = pl.ds(base + j * _IDXW, _IDXW)
            pltpu.sync_copy(ubufs[j % 2], ue_hbm.at[dst])
            pltpu.sync_copy(mbufs[j % 2], me_hbm.at[dst])

    return gather


def _unpack_sel(g, half, side):
    f32 = jnp.float32
    hi = lax.bitcast_convert_type(g & jnp.int32(-65536), f32)
    lo = lax.bitcast_convert_type(g << 16, f32)
    plane = jnp.where(half[...] > 0.5, lo, hi)
    lane = lax.broadcasted_iota(jnp.int32, (_TB, _DP), 1)
    left = (lane < _D).astype(f32)
    smask = jnp.where(side[...] > 0.5, 1.0 - left, left)
    return plane * smask


def _mlp_body(ue, me, uh, usd, mh, msd, w1u, w1m, b1, w2, b2, w3, b3, w4, b4,
              w5, b5, out):
    f32 = jnp.float32
    x = jnp.dot(_unpack_sel(ue[...], uh, usd), w1u[...],
                preferred_element_type=f32)
    x = x + jnp.dot(_unpack_sel(me[...], mh, msd), w1m[...],
                    preferred_element_type=f32)
    x = jnp.maximum(x + b1[...], 0.0)
    x = jnp.maximum(jnp.dot(x, w2[...], preferred_element_type=f32) + b2[...], 0.0)
    x = jnp.maximum(jnp.dot(x, w3[...], preferred_element_type=f32) + b3[...], 0.0)
    x = jnp.maximum(jnp.dot(x, w4[...], preferred_element_type=f32) + b4[...], 0.0)
    out[...] = jnp.dot(x, w5[...], preferred_element_type=f32) + b5[...]


def _full(shape):
    return pl.BlockSpec(shape, lambda i: (0,) * len(shape))


def _mlp(ue, me, uh, usd, mh, msd,
         w1u, w1m, b1, w2, b2, w3, b3, w4, b4, w5, b5):
    nblk = _B // _TB
    row = pl.BlockSpec((_TB, _DP), lambda i: (i, 0))
    col = pl.BlockSpec((_TB, 1), lambda i: (i, 0))
    return pl.pallas_call(
        _mlp_body,
        grid=(nblk,),
        in_specs=[
            row, row, col, col, col, col,
            _full(w1u.shape), _full(w1m.shape), _full(b1.shape),
            _full(w2.shape), _full(b2.shape),
            _full(w3.shape), _full(b3.shape),
            _full(w4.shape), _full(b4.shape),
            _full(w5.shape), _full(b5.shape),
        ],
        out_specs=pl.BlockSpec((_TB, 1), lambda i: (i, 0)),
        out_shape=jax.ShapeDtypeStruct((_B, 1), jnp.float32),
    )(ue, me, uh, usd, mh, msd,
      w1u, w1m, b1, w2, b2, w3, b3, w4, b4, w5, b5)


def kernel(user, movie, user_table, movie_table,
           W1, b1, W2, b2, W3, b3, W4, b4, W5, b5):
    user = user.astype(jnp.int32)
    movie = movie.astype(jnp.int32)
    urow = (((user >> (_QSH + 2)) << _QSH) + (user & (_Q - 1))).reshape(
        _NW, _NCHUNK, _IDXW)
    mrow = (((movie >> (_QSH + 2)) << _QSH) + (movie & (_Q - 1))).reshape(
        _NW, _NCHUNK, _IDXW)
    f32 = jnp.float32
    usd = ((user >> (_QSH + 1)) & 1).astype(f32).reshape(_B, 1)
    msd = ((movie >> (_QSH + 1)) & 1).astype(f32).reshape(_B, 1)
    uh = ((user >> _QSH) & 1).astype(f32).reshape(_B, 1)
    mh = ((movie >> _QSH) & 1).astype(f32).reshape(_B, 1)
    tu, tm = _convert(user_table.T, movie_table.T)
    ue, me = _build_gather()(urow, mrow, tu, tm)
    w1u = jnp.concatenate([W1[:_D], W1[:_D]], axis=0)
    w1m = jnp.concatenate([W1[_D:], W1[_D:]], axis=0)
    return _mlp(
        ue, me, uh, usd, mh, msd,
        w1u, w1m, b1.reshape(1, -1),
        W2, b2.reshape(1, -1),
        W3, b3.reshape(1, -1),
        W4, b4.reshape(1, -1),
        W5, b5.reshape(1, -1),
    )
